# baseline (speedup 1.0000x reference)
import functools

import jax
import jax.numpy as jnp
from jax.experimental import pallas as pl
from jax.experimental.pallas import tpu as pltpu


def _make_body(ksplit, kw):
    def body(*refs):
        x_k_ref, x_i_ref = refs[0], refs[1]
        a0_refs = refs[2:2 + ksplit]
        a1_refs = refs[2 + ksplit:2 + 2 * ksplit]
        ws_ref, w0_ref, w1_ref, b_ref, o_ref, y0_s, y1_s = refs[2 + 2 * ksplit:]
        i = pl.program_id(0)

        @pl.when(i == 0)
        def _compute_y():
            xk = x_k_ref[...]
            y0_s[...] = jnp.dot(xk, w0_ref[...], preferred_element_type=jnp.float32)
            y1_s[...] = jnp.dot(xk, w1_ref[...], preferred_element_type=jnp.float32)

        acc = jnp.dot(x_i_ref[...], ws_ref[...],
                      preferred_element_type=jnp.float32) + b_ref[...]
        for s in range(ksplit):
            acc += jnp.dot(a0_refs[s][...], y0_s[s * kw:(s + 1) * kw, :],
                           preferred_element_type=jnp.float32)
            acc += jnp.dot(a1_refs[s][...], y1_s[s * kw:(s + 1) * kw, :],
                           preferred_element_type=jnp.float32)
        o_ref[...] = jnp.maximum(acc, 0.0)

    return body


@functools.partial(jax.jit, static_argnames=("tm", "ksplit"))
def _rgcn(x, a0, a1, ws, w0, w1, b, tm=256, ksplit=2):
    n, f = x.shape
    u = ws.shape[1]
    ni = n // tm
    kw = n // ksplit
    b2 = b.reshape(1, u)

    def a_spec(s):
        return pl.BlockSpec((tm, kw), lambda i, s=s: (i, s))

    out = pl.pallas_call(
        _make_body(ksplit, kw),
        grid=(ni,),
        in_specs=[
            pl.BlockSpec((n, f), lambda i: (0, 0)),
            pl.BlockSpec((tm, f), lambda i: (i, 0)),
            *[a_spec(s) for s in range(ksplit)],
            *[a_spec(s) for s in range(ksplit)],
            pl.BlockSpec((f, u), lambda i: (0, 0)),
            pl.BlockSpec((f, u), lambda i: (0, 0)),
            pl.BlockSpec((f, u), lambda i: (0, 0)),
            pl.BlockSpec((1, u), lambda i: (0, 0)),
        ],
        out_specs=pl.BlockSpec((tm, u), lambda i: (i, 0)),
        out_shape=jax.ShapeDtypeStruct((n, u), jnp.float32),
        scratch_shapes=[
            pltpu.VMEM((n, u), jnp.float32),
            pltpu.VMEM((n, u), jnp.float32),
        ],
        compiler_params=pltpu.CompilerParams(
            dimension_semantics=("arbitrary",)),
    )(x, x, *([a0] * ksplit), *([a1] * ksplit), ws, w0, w1, b2)
    return out


def kernel(features, A_0, A_1, self_kernel, rel_kernel_0, rel_kernel_1, bias):
    x = features[0]
    out = _rgcn(x, A_0, A_1, self_kernel, rel_kernel_0, rel_kernel_1, bias)
    return out[None, ...]
